# baseline (speedup 1.0000x reference)
import functools

import jax
import jax.numpy as jnp
from jax import lax
from jax.experimental import pallas as pl
from jax.experimental.pallas import tpu as pltpu
from jax.experimental.pallas import tpu_sc as plsc

B, S, D = 1, 2048, 768
E, K, F = 64, 2, 128
EPS = 1e-6
T = B * S

G = 128
NT = 96
NTG = NT * G
NC, NS = 2, 16
NW = NC * NS
L = 16


def _route_body(x_ref, lnw_ref, rw_ref, h_ref, i1_ref, i2_ref, w1_ref, w2_ref):
    x = x_ref[...]
    v = jnp.mean(x * x, axis=-1, keepdims=True)
    h = x * jax.lax.rsqrt(v + EPS) * lnw_ref[...]
    h_ref[...] = h
    logits = jnp.dot(h, rw_ref[...], preferred_element_type=jnp.float32)
    iota = jax.lax.broadcasted_iota(jnp.int32, logits.shape, 1)
    m1 = jnp.max(logits, axis=-1, keepdims=True)
    i1 = jnp.min(jnp.where(logits == m1, iota, E), axis=-1, keepdims=True)
    l2 = jnp.where(iota == i1, -jnp.inf, logits)
    m2 = jnp.max(l2, axis=-1, keepdims=True)
    i2 = jnp.min(jnp.where(l2 == m2, iota, E), axis=-1, keepdims=True)
    e2 = jnp.exp(m2 - m1)
    s = 1.0 + e2
    i1_ref[...] = i1
    i2_ref[...] = i2
    w1_ref[...] = 1.0 / s
    w2_ref[...] = e2 / s


def _route(x, ln_weight, router_w):
    return pl.pallas_call(
        _route_body,
        out_shape=(
            jax.ShapeDtypeStruct((T, D), jnp.float32),
            jax.ShapeDtypeStruct((T, 1), jnp.int32),
            jax.ShapeDtypeStruct((T, 1), jnp.int32),
            jax.ShapeDtypeStruct((T, 1), jnp.float32),
            jax.ShapeDtypeStruct((T, 1), jnp.float32),
        ),
    )(x, ln_weight.reshape(1, D), router_w)


def _metadata(i1, i2, w1, w2):
    i12 = jnp.concatenate([i1, i2], axis=1)
    w12 = jnp.concatenate([w1, w2], axis=1)
    oh = i12[:, :, None] == jnp.arange(E, dtype=jnp.int32)[None, None, :]
    A = oh.sum(1).astype(jnp.int32)
    cum = jnp.cumsum(A, axis=0)
    counts = cum[-1]
    cum_excl = cum - A
    pos = jnp.sum(jnp.where(oh, cum_excl[:, None, :], 0), axis=2)
    tiles = (counts + G - 1) // G
    na = jnp.sum(tiles).astype(jnp.int32)
    tile_start = jnp.concatenate(
        [jnp.zeros((1,), jnp.int32),
         jnp.cumsum(tiles)[:-1].astype(jnp.int32)])
    row = jnp.take(tile_start * G, i12) + pos
    rowf = row.reshape(-1)
    tok = jnp.repeat(jnp.arange(T, dtype=jnp.int32), 2)
    src = jnp.zeros((NTG,), jnp.int32).at[rowf].set(tok)
    wsort = jnp.zeros((NTG,), jnp.float32).at[rowf].set(w12.reshape(-1))
    idx_t = jnp.arange(NT, dtype=jnp.int32)
    te = jnp.searchsorted(tile_start, idx_t, side="right").astype(jnp.int32) - 1
    te = jnp.clip(te, 0, E - 1)
    te = jnp.where(idx_t < na, te, jnp.take(te, na - 1))
    te_ext = jnp.concatenate([te, na[None]])
    return src, wsort.reshape(NTG, 1), te_ext, row[:, 0], row[:, 1]


def _gather_body(h_hbm, src_hbm, x_hbm, idx_v, rows_v, sem):
    wid = lax.axis_index("s") * NC + lax.axis_index("c")
    per_w = NTG // NW
    ch = 128
    base = wid * per_w
    for c in range(per_w // ch):
        b = base + c * ch
        pltpu.sync_copy(src_hbm.at[pl.ds(b, ch)], idx_v)
        pltpu.async_copy(h_hbm.at[idx_v], rows_v, sem).wait()
        pltpu.sync_copy(rows_v, x_hbm.at[pl.ds(b, ch)])


def _gather(h, src):
    mesh = plsc.VectorSubcoreMesh(
        core_axis_name="c", subcore_axis_name="s", num_cores=NC,
        num_subcores=NS)
    f = pl.kernel(
        _gather_body,
        out_type=jax.ShapeDtypeStruct((NTG, D), jnp.float32),
        mesh=mesh,
        scratch_types=[
            pltpu.VMEM((128,), jnp.int32),
            pltpu.VMEM((128, D), jnp.float32),
            pltpu.SemaphoreType.DMA,
        ],
    )
    return f(h, src)


def _gmm_body(te_ref, x_ref, w_ref, wg_ref, wu_ref, wd_ref, y_ref):
    i = pl.program_id(0)

    @pl.when(i < te_ref[NT])
    def _():
        x = x_ref[...]
        g = jnp.dot(x, wg_ref[0], preferred_element_type=jnp.float32)
        u = jnp.dot(x, wu_ref[0], preferred_element_type=jnp.float32)
        a = (g * jax.nn.sigmoid(g)) * u
        y = jnp.dot(a, wd_ref[0], preferred_element_type=jnp.float32)
        y_ref[...] = w_ref[...] * y


def _gmm(te_ext, X, wsort, W_gate, W_up, W_down):
    def _rows(i, te):
        return (jnp.minimum(i, te[NT] - 1), 0)

    def _widx(i, te):
        return (te[jnp.minimum(i, te[NT] - 1)], 0, 0)

    grid_spec = pltpu.PrefetchScalarGridSpec(
        num_scalar_prefetch=1,
        grid=(NT,),
        in_specs=[
            pl.BlockSpec((G, D), _rows),
            pl.BlockSpec((G, 1), _rows),
            pl.BlockSpec((1, D, F), _widx),
            pl.BlockSpec((1, D, F), _widx),
            pl.BlockSpec((1, F, D), _widx),
        ],
        out_specs=pl.BlockSpec((G, D), _rows),
    )
    return pl.pallas_call(
        _gmm_body,
        grid_spec=grid_spec,
        out_shape=jax.ShapeDtypeStruct((NTG, D), jnp.float32),
    )(te_ext, X, wsort, W_gate, W_up, W_down)


def _combine_body(y_hbm, r0_hbm, r1_hbm, res_hbm, out_hbm,
                  idx0, idx1, b0, b1, br, sem0, sem1, semr):
    wid = lax.axis_index("s") * NC + lax.axis_index("c")
    per_w = T // NW
    for sub in range(per_w // L):
        t0 = wid * per_w + sub * L
        pltpu.sync_copy(r0_hbm.at[pl.ds(t0, L)], idx0)
        pltpu.sync_copy(r1_hbm.at[pl.ds(t0, L)], idx1)
        cp0 = pltpu.async_copy(y_hbm.at[idx0], b0, sem0)
        cp1 = pltpu.async_copy(y_hbm.at[idx1], b1, sem1)
        cpr = pltpu.async_copy(res_hbm.at[pl.ds(t0, L)], br, semr)
        cp0.wait()
        cp1.wait()
        cpr.wait()

        def col(c, r):
            s = pl.ds(c * L, L)
            br[r, s] = br[r, s] + b0[r, s] + b1[r, s]
            return r

        def rowloop(r, _):
            lax.fori_loop(0, D // L, col, r)
            return 0

        lax.fori_loop(0, L, rowloop, 0)
        pltpu.sync_copy(br, out_hbm.at[pl.ds(t0, L)])


def _combine(Y, r0, r1, residual):
    mesh = plsc.VectorSubcoreMesh(
        core_axis_name="c", subcore_axis_name="s", num_cores=NC,
        num_subcores=NS)
    f = pl.kernel(
        _combine_body,
        out_type=jax.ShapeDtypeStruct((T, D), jnp.float32),
        mesh=mesh,
        scratch_types=[
            pltpu.VMEM((L,), jnp.int32),
            pltpu.VMEM((L,), jnp.int32),
            pltpu.VMEM((L, D), jnp.float32),
            pltpu.VMEM((L, D), jnp.float32),
            pltpu.VMEM((L, D), jnp.float32),
            pltpu.SemaphoreType.DMA,
            pltpu.SemaphoreType.DMA,
            pltpu.SemaphoreType.DMA,
        ],
    )
    return f(Y, r0, r1, residual)


def kernel(hidden_states, attention_mask, position_ids, ln_weight, router_w,
           W_gate, W_up, W_down):
    x = hidden_states.reshape(T, D)
    h, i1, i2, w1, w2 = _route(x, ln_weight, router_w)
    src, wsort, te_ext, r0, r1 = _metadata(i1, i2, w1, w2)
    X = _gather(h, src)
    Y = _gmm(te_ext, X, wsort, W_gate, W_up, W_down)
    out = _combine(Y, r0, r1, x)
    return out.reshape(B, S, D)


# speedup vs baseline: 1.3747x; 1.3747x over previous
import functools

import jax
import jax.numpy as jnp
from jax import lax
from jax.experimental import pallas as pl
from jax.experimental.pallas import tpu as pltpu
from jax.experimental.pallas import tpu_sc as plsc

B, S, D = 1, 2048, 768
E, K, F = 64, 2, 128
EPS = 1e-6
T = B * S

G = 64
NT = 128
NTG = NT * G
NC, NS = 2, 16
NW = NC * NS
L = 16


def _route_body(x_ref, lnw_ref, rw_ref, h_ref, i1_ref, i2_ref, w1_ref, w2_ref):
    x = x_ref[...]
    v = jnp.mean(x * x, axis=-1, keepdims=True)
    h = x * jax.lax.rsqrt(v + EPS) * lnw_ref[...]
    h_ref[...] = h
    logits = jnp.dot(h, rw_ref[...], preferred_element_type=jnp.float32)
    iota = jax.lax.broadcasted_iota(jnp.int32, logits.shape, 1)
    m1 = jnp.max(logits, axis=-1, keepdims=True)
    i1 = jnp.min(jnp.where(logits == m1, iota, E), axis=-1, keepdims=True)
    l2 = jnp.where(iota == i1, -jnp.inf, logits)
    m2 = jnp.max(l2, axis=-1, keepdims=True)
    i2 = jnp.min(jnp.where(l2 == m2, iota, E), axis=-1, keepdims=True)
    e2 = jnp.exp(m2 - m1)
    s = 1.0 + e2
    i1_ref[...] = i1
    i2_ref[...] = i2
    w1_ref[...] = 1.0 / s
    w2_ref[...] = e2 / s


def _route(x, ln_weight, router_w):
    return pl.pallas_call(
        _route_body,
        out_shape=(
            jax.ShapeDtypeStruct((T, D), jnp.float32),
            jax.ShapeDtypeStruct((T, 1), jnp.int32),
            jax.ShapeDtypeStruct((T, 1), jnp.int32),
            jax.ShapeDtypeStruct((T, 1), jnp.float32),
            jax.ShapeDtypeStruct((T, 1), jnp.float32),
        ),
    )(x, ln_weight.reshape(1, D), router_w)


def _metadata(i1, i2, w1, w2):
    i12 = jnp.concatenate([i1, i2], axis=1)
    w12 = jnp.concatenate([w1, w2], axis=1)
    oh = i12[:, :, None] == jnp.arange(E, dtype=jnp.int32)[None, None, :]
    A = oh.sum(1).astype(jnp.int32)
    cum = jnp.cumsum(A, axis=0)
    counts = cum[-1]
    cum_excl = cum - A
    pos = jnp.sum(jnp.where(oh, cum_excl[:, None, :], 0), axis=2)
    tiles = (counts + G - 1) // G
    na = jnp.sum(tiles).astype(jnp.int32)
    tile_start = jnp.concatenate(
        [jnp.zeros((1,), jnp.int32),
         jnp.cumsum(tiles)[:-1].astype(jnp.int32)])
    row = jnp.take(tile_start * G, i12) + pos
    rowf = row.reshape(-1)
    tok = jnp.repeat(jnp.arange(T, dtype=jnp.int32), 2)
    src = jnp.zeros((NTG,), jnp.int32).at[rowf].set(tok)
    wsort = jnp.zeros((NTG,), jnp.float32).at[rowf].set(w12.reshape(-1))
    idx_t = jnp.arange(NT, dtype=jnp.int32)
    te = jnp.searchsorted(tile_start, idx_t, side="right").astype(jnp.int32) - 1
    te = jnp.clip(te, 0, E - 1)
    te = jnp.where(idx_t < na, te, jnp.take(te, na - 1))
    te_ext = jnp.concatenate([te, na[None]])
    return src, wsort.reshape(NTG, 1), te_ext, row[:, 0], row[:, 1]


_GCH = 64
_GNCH = (NTG // NW) // _GCH


def _gather_body(h_hbm, src_hbm, x_hbm, idx0, idx1, buf0, buf1,
                 gs0, gs1, os0, os1):
    wid = lax.axis_index("s") * NC + lax.axis_index("c")
    base = wid * (NTG // NW)
    idx = (idx0, idx1)
    buf = (buf0, buf1)
    gs = (gs0, gs1)
    osem = (os0, os1)

    def issue(c):
        b = base + c * _GCH
        pltpu.sync_copy(src_hbm.at[pl.ds(b, _GCH)], idx[c % 2])
        return pltpu.async_copy(h_hbm.at[idx[c % 2]], buf[c % 2], gs[c % 2])

    gh = {0: issue(0)}
    oh = {}
    for c in range(_GNCH):
        if c + 1 < _GNCH:
            if c - 1 >= 0:
                oh[c - 1].wait()
            gh[c + 1] = issue(c + 1)
        gh[c].wait()
        oh[c] = pltpu.async_copy(
            buf[c % 2], x_hbm.at[pl.ds(base + c * _GCH, _GCH)], osem[c % 2])
    oh[_GNCH - 2].wait()
    oh[_GNCH - 1].wait()


def _gather(h, src):
    mesh = plsc.VectorSubcoreMesh(
        core_axis_name="c", subcore_axis_name="s", num_cores=NC,
        num_subcores=NS)
    f = pl.kernel(
        _gather_body,
        out_type=jax.ShapeDtypeStruct((NTG, D), jnp.float32),
        mesh=mesh,
        scratch_types=[
            pltpu.VMEM((_GCH,), jnp.int32),
            pltpu.VMEM((_GCH,), jnp.int32),
            pltpu.VMEM((_GCH, D), jnp.float32),
            pltpu.VMEM((_GCH, D), jnp.float32),
            pltpu.SemaphoreType.DMA,
            pltpu.SemaphoreType.DMA,
            pltpu.SemaphoreType.DMA,
            pltpu.SemaphoreType.DMA,
        ],
    )
    return f(h, src)


def _gmm_body(te_ref, x_ref, w_ref, wg_ref, wu_ref, wd_ref, y_ref):
    i = pl.program_id(0)

    @pl.when(i < te_ref[NT])
    def _():
        x = x_ref[...]
        g = jnp.dot(x, wg_ref[0], preferred_element_type=jnp.float32)
        u = jnp.dot(x, wu_ref[0], preferred_element_type=jnp.float32)
        a = (g * jax.nn.sigmoid(g)) * u
        y = jnp.dot(a, wd_ref[0], preferred_element_type=jnp.float32)
        y_ref[...] = w_ref[...] * y


def _gmm(te_ext, X, wsort, W_gate, W_up, W_down):
    def _rows(i, te):
        return (jnp.minimum(i, te[NT] - 1), 0)

    def _widx(i, te):
        return (te[jnp.minimum(i, te[NT] - 1)], 0, 0)

    grid_spec = pltpu.PrefetchScalarGridSpec(
        num_scalar_prefetch=1,
        grid=(NT,),
        in_specs=[
            pl.BlockSpec((G, D), _rows),
            pl.BlockSpec((G, 1), _rows),
            pl.BlockSpec((1, D, F), _widx),
            pl.BlockSpec((1, D, F), _widx),
            pl.BlockSpec((1, F, D), _widx),
        ],
        out_specs=pl.BlockSpec((G, D), _rows),
    )
    return pl.pallas_call(
        _gmm_body,
        grid_spec=grid_spec,
        out_shape=jax.ShapeDtypeStruct((NTG, D), jnp.float32),
    )(te_ext, X, wsort, W_gate, W_up, W_down)


def _combine_body(y_hbm, r0_hbm, r1_hbm, res_hbm, out_hbm,
                  idx0, idx1, b0, b1, br, sem0, sem1, semr):
    wid = lax.axis_index("s") * NC + lax.axis_index("c")
    per_w = T // NW
    for sub in range(per_w // L):
        t0 = wid * per_w + sub * L
        pltpu.sync_copy(r0_hbm.at[pl.ds(t0, L)], idx0)
        pltpu.sync_copy(r1_hbm.at[pl.ds(t0, L)], idx1)
        cp0 = pltpu.async_copy(y_hbm.at[idx0], b0, sem0)
        cp1 = pltpu.async_copy(y_hbm.at[idx1], b1, sem1)
        cpr = pltpu.async_copy(res_hbm.at[pl.ds(t0, L)], br, semr)
        cp0.wait()
        cp1.wait()
        cpr.wait()

        def col(c, r):
            s = pl.ds(c * L, L)
            br[r, s] = br[r, s] + b0[r, s] + b1[r, s]
            return r

        def rowloop(r, _):
            lax.fori_loop(0, D // L, col, r)
            return 0

        lax.fori_loop(0, L, rowloop, 0)
        pltpu.sync_copy(br, out_hbm.at[pl.ds(t0, L)])


def _combine(Y, r0, r1, residual):
    mesh = plsc.VectorSubcoreMesh(
        core_axis_name="c", subcore_axis_name="s", num_cores=NC,
        num_subcores=NS)
    f = pl.kernel(
        _combine_body,
        out_type=jax.ShapeDtypeStruct((T, D), jnp.float32),
        mesh=mesh,
        scratch_types=[
            pltpu.VMEM((L,), jnp.int32),
            pltpu.VMEM((L,), jnp.int32),
            pltpu.VMEM((L, D), jnp.float32),
            pltpu.VMEM((L, D), jnp.float32),
            pltpu.VMEM((L, D), jnp.float32),
            pltpu.SemaphoreType.DMA,
            pltpu.SemaphoreType.DMA,
            pltpu.SemaphoreType.DMA,
        ],
    )
    return f(Y, r0, r1, residual)


def kernel(hidden_states, attention_mask, position_ids, ln_weight, router_w,
           W_gate, W_up, W_down):
    x = hidden_states.reshape(T, D)
    h, i1, i2, w1, w2 = _route(x, ln_weight, router_w)
    src, wsort, te_ext, r0, r1 = _metadata(i1, i2, w1, w2)
    X = _gather(h, src)
    Y = _gmm(te_ext, X, wsort, W_gate, W_up, W_down)
    out = _combine(Y, r0, r1, x)
    return out.reshape(B, S, D)


# speedup vs baseline: 1.3940x; 1.0140x over previous
import functools

import jax
import jax.numpy as jnp
from jax import lax
from jax.experimental import pallas as pl
from jax.experimental.pallas import tpu as pltpu
from jax.experimental.pallas import tpu_sc as plsc

B, S, D = 1, 2048, 768
E, K, F = 64, 2, 128
EPS = 1e-6
T = B * S

G = 64
NT = 128
NTG = NT * G
NC, NS = 2, 16
NW = NC * NS
L = 16


def _route_body(x_ref, lnw_ref, rw_ref, h_ref, i1_ref, i2_ref, w1_ref, w2_ref):
    x = x_ref[...]
    v = jnp.mean(x * x, axis=-1, keepdims=True)
    h = x * jax.lax.rsqrt(v + EPS) * lnw_ref[...]
    h_ref[...] = h
    logits = jnp.dot(h, rw_ref[...], preferred_element_type=jnp.float32)
    iota = jax.lax.broadcasted_iota(jnp.int32, logits.shape, 1)
    m1 = jnp.max(logits, axis=-1, keepdims=True)
    i1 = jnp.min(jnp.where(logits == m1, iota, E), axis=-1, keepdims=True)
    l2 = jnp.where(iota == i1, -jnp.inf, logits)
    m2 = jnp.max(l2, axis=-1, keepdims=True)
    i2 = jnp.min(jnp.where(l2 == m2, iota, E), axis=-1, keepdims=True)
    e2 = jnp.exp(m2 - m1)
    s = 1.0 + e2
    i1_ref[...] = i1
    i2_ref[...] = i2
    w1_ref[...] = 1.0 / s
    w2_ref[...] = e2 / s


def _route(x, ln_weight, router_w):
    return pl.pallas_call(
        _route_body,
        out_shape=(
            jax.ShapeDtypeStruct((T, D), jnp.float32),
            jax.ShapeDtypeStruct((T, 1), jnp.int32),
            jax.ShapeDtypeStruct((T, 1), jnp.int32),
            jax.ShapeDtypeStruct((T, 1), jnp.float32),
            jax.ShapeDtypeStruct((T, 1), jnp.float32),
        ),
    )(x, ln_weight.reshape(1, D), router_w)


def _metadata(i1, i2, w1, w2):
    i12 = jnp.concatenate([i1, i2], axis=1)
    w12 = jnp.concatenate([w1, w2], axis=1)
    oh = i12[:, :, None] == jnp.arange(E, dtype=jnp.int32)[None, None, :]
    A = oh.sum(1).astype(jnp.int32)
    cum = jnp.cumsum(A, axis=0)
    counts = cum[-1]
    cum_excl = cum - A
    pos = jnp.sum(jnp.where(oh, cum_excl[:, None, :], 0), axis=2)
    tiles = (counts + G - 1) // G
    na = jnp.sum(tiles).astype(jnp.int32)
    tile_start = jnp.concatenate(
        [jnp.zeros((1,), jnp.int32),
         jnp.cumsum(tiles)[:-1].astype(jnp.int32)])
    row = jnp.take(tile_start * G, i12) + pos
    rowf = row.reshape(-1)
    tok = jnp.repeat(jnp.arange(T, dtype=jnp.int32), 2)
    src = jnp.zeros((NTG,), jnp.int32).at[rowf].set(tok)
    wsort = jnp.zeros((NTG,), jnp.float32).at[rowf].set(w12.reshape(-1))
    idx_t = jnp.arange(NT, dtype=jnp.int32)
    te = jnp.searchsorted(tile_start, idx_t, side="right").astype(jnp.int32) - 1
    te = jnp.clip(te, 0, E - 1)
    te = jnp.where(idx_t < na, te, jnp.take(te, na - 1))
    te_ext = jnp.concatenate([te, na[None]])
    return src, wsort.reshape(NTG, 1), te_ext, row[:, 0], row[:, 1]


_GCH = 64
_GNCH = (NTG // NW) // _GCH


def _gather_body(h_hbm, src_hbm, x_hbm, idx0, idx1, buf0, buf1,
                 gs0, gs1, os0, os1):
    wid = lax.axis_index("s") * NC + lax.axis_index("c")
    base = wid * (NTG // NW)
    idx = (idx0, idx1)
    buf = (buf0, buf1)
    gs = (gs0, gs1)
    osem = (os0, os1)

    def issue(c):
        b = base + c * _GCH
        pltpu.sync_copy(src_hbm.at[pl.ds(b, _GCH)], idx[c % 2])
        return [
            pltpu.async_copy(
                h_hbm.at[idx[c % 2].at[pl.ds(q * L, L)]],
                buf[c % 2].at[pl.ds(q * L, L)],
                gs[c % 2])
            for q in range(_GCH // L)
        ]

    gh = {0: issue(0)}
    oh = {}
    for c in range(_GNCH):
        if c + 1 < _GNCH:
            if c - 1 >= 0:
                oh[c - 1].wait()
            gh[c + 1] = issue(c + 1)
        for hh in gh[c]:
            hh.wait()
        oh[c] = pltpu.async_copy(
            buf[c % 2], x_hbm.at[pl.ds(base + c * _GCH, _GCH)], osem[c % 2])
    oh[_GNCH - 2].wait()
    oh[_GNCH - 1].wait()


def _gather(h, src):
    mesh = plsc.VectorSubcoreMesh(
        core_axis_name="c", subcore_axis_name="s", num_cores=NC,
        num_subcores=NS)
    f = pl.kernel(
        _gather_body,
        out_type=jax.ShapeDtypeStruct((NTG, D), jnp.float32),
        mesh=mesh,
        scratch_types=[
            pltpu.VMEM((_GCH,), jnp.int32),
            pltpu.VMEM((_GCH,), jnp.int32),
            pltpu.VMEM((_GCH, D), jnp.float32),
            pltpu.VMEM((_GCH, D), jnp.float32),
            pltpu.SemaphoreType.DMA,
            pltpu.SemaphoreType.DMA,
            pltpu.SemaphoreType.DMA,
            pltpu.SemaphoreType.DMA,
        ],
    )
    return f(h, src)


def _gmm_body(te_ref, x_ref, w_ref, wg_ref, wu_ref, wd_ref, y_ref):
    i = pl.program_id(0)

    @pl.when(i < te_ref[NT])
    def _():
        x = x_ref[...]
        g = jnp.dot(x, wg_ref[0], preferred_element_type=jnp.float32)
        u = jnp.dot(x, wu_ref[0], preferred_element_type=jnp.float32)
        a = (g * jax.nn.sigmoid(g)) * u
        y = jnp.dot(a, wd_ref[0], preferred_element_type=jnp.float32)
        y_ref[...] = w_ref[...] * y


def _gmm(te_ext, X, wsort, W_gate, W_up, W_down):
    def _rows(i, te):
        return (jnp.minimum(i, te[NT] - 1), 0)

    def _widx(i, te):
        return (te[jnp.minimum(i, te[NT] - 1)], 0, 0)

    grid_spec = pltpu.PrefetchScalarGridSpec(
        num_scalar_prefetch=1,
        grid=(NT,),
        in_specs=[
            pl.BlockSpec((G, D), _rows),
            pl.BlockSpec((G, 1), _rows),
            pl.BlockSpec((1, D, F), _widx),
            pl.BlockSpec((1, D, F), _widx),
            pl.BlockSpec((1, F, D), _widx),
        ],
        out_specs=pl.BlockSpec((G, D), _rows),
    )
    return pl.pallas_call(
        _gmm_body,
        grid_spec=grid_spec,
        out_shape=jax.ShapeDtypeStruct((NTG, D), jnp.float32),
    )(te_ext, X, wsort, W_gate, W_up, W_down)


_CSUB = (T // NW) // L


def _combine_body(y_hbm, r0_hbm, r1_hbm, res_hbm, out_hbm,
                  idx0a, idx0b, idx1a, idx1b, b0a, b0b, b1a, b1b, bra, brb,
                  sg0a, sg0b, sg1a, sg1b, sra, srb, soa, sob):
    wid = lax.axis_index("s") * NC + lax.axis_index("c")
    per_w = T // NW
    idx0 = (idx0a, idx0b)
    idx1 = (idx1a, idx1b)
    b0 = (b0a, b0b)
    b1 = (b1a, b1b)
    br = (bra, brb)
    sg0 = (sg0a, sg0b)
    sg1 = (sg1a, sg1b)
    sr = (sra, srb)
    so = (soa, sob)

    def issue(s):
        p = s % 2
        t0 = wid * per_w + s * L
        pltpu.sync_copy(r0_hbm.at[pl.ds(t0, L)], idx0[p])
        pltpu.sync_copy(r1_hbm.at[pl.ds(t0, L)], idx1[p])
        return [
            pltpu.async_copy(y_hbm.at[idx0[p]], b0[p], sg0[p]),
            pltpu.async_copy(y_hbm.at[idx1[p]], b1[p], sg1[p]),
            pltpu.async_copy(res_hbm.at[pl.ds(t0, L)], br[p], sr[p]),
        ]

    gh = {0: issue(0)}
    oh = {}
    for s in range(_CSUB):
        p = s % 2
        if s + 1 < _CSUB:
            if s - 1 >= 0:
                oh[s - 1].wait()
            gh[s + 1] = issue(s + 1)
        for hh in gh[s]:
            hh.wait()

        def col(c, r):
            sl = pl.ds(c * L, L)
            br[p][r, sl] = br[p][r, sl] + b0[p][r, sl] + b1[p][r, sl]
            return r

        def rowloop(r, _):
            lax.fori_loop(0, D // L, col, r, unroll=8)
            return 0

        lax.fori_loop(0, L, rowloop, 0)
        oh[s] = pltpu.async_copy(
            br[p], out_hbm.at[pl.ds(wid * per_w + s * L, L)], so[p])
    oh[_CSUB - 2].wait()
    oh[_CSUB - 1].wait()


def _combine(Y, r0, r1, residual):
    mesh = plsc.VectorSubcoreMesh(
        core_axis_name="c", subcore_axis_name="s", num_cores=NC,
        num_subcores=NS)
    f = pl.kernel(
        _combine_body,
        out_type=jax.ShapeDtypeStruct((T, D), jnp.float32),
        mesh=mesh,
        scratch_types=(
            [pltpu.VMEM((L,), jnp.int32)] * 4
            + [pltpu.VMEM((L, D), jnp.float32)] * 6
            + [pltpu.SemaphoreType.DMA] * 8
        ),
    )
    return f(Y, r0, r1, residual)


def kernel(hidden_states, attention_mask, position_ids, ln_weight, router_w,
           W_gate, W_up, W_down):
    x = hidden_states.reshape(T, D)
    h, i1, i2, w1, w2 = _route(x, ln_weight, router_w)
    src, wsort, te_ext, r0, r1 = _metadata(i1, i2, w1, w2)
    X = _gather(h, src)
    Y = _gmm(te_ext, X, wsort, W_gate, W_up, W_down)
    out = _combine(Y, r0, r1, x)
    return out.reshape(B, S, D)


# speedup vs baseline: 2.3597x; 1.6928x over previous
import functools

import jax
import jax.numpy as jnp
from jax import lax
from jax.experimental import pallas as pl
from jax.experimental.pallas import tpu as pltpu
from jax.experimental.pallas import tpu_sc as plsc

B, S, D = 1, 2048, 768
E, K, F = 64, 2, 128
EPS = 1e-6
T = B * S

G = 64
NT = 128
NTG = NT * G
NC, NS = 2, 16
NW = NC * NS
L = 16


def _route_body(x_ref, lnw_ref, rw_ref, h_ref, i1_ref, i2_ref, w1_ref, w2_ref):
    x = x_ref[...]
    v = jnp.mean(x * x, axis=-1, keepdims=True)
    h = x * jax.lax.rsqrt(v + EPS) * lnw_ref[...]
    h_ref[...] = h
    logits = jnp.dot(h, rw_ref[...], preferred_element_type=jnp.float32)
    iota = jax.lax.broadcasted_iota(jnp.int32, logits.shape, 1)
    m1 = jnp.max(logits, axis=-1, keepdims=True)
    i1 = jnp.min(jnp.where(logits == m1, iota, E), axis=-1, keepdims=True)
    l2 = jnp.where(iota == i1, -jnp.inf, logits)
    m2 = jnp.max(l2, axis=-1, keepdims=True)
    i2 = jnp.min(jnp.where(l2 == m2, iota, E), axis=-1, keepdims=True)
    e2 = jnp.exp(m2 - m1)
    s = 1.0 + e2
    i1_ref[...] = i1
    i2_ref[...] = i2
    w1_ref[...] = 1.0 / s
    w2_ref[...] = e2 / s


def _route(x, ln_weight, router_w):
    return pl.pallas_call(
        _route_body,
        out_shape=(
            jax.ShapeDtypeStruct((T, D), jnp.float32),
            jax.ShapeDtypeStruct((T, 1), jnp.int32),
            jax.ShapeDtypeStruct((T, 1), jnp.int32),
            jax.ShapeDtypeStruct((T, 1), jnp.float32),
            jax.ShapeDtypeStruct((T, 1), jnp.float32),
        ),
    )(x, ln_weight.reshape(1, D), router_w)


def _metadata(i1, i2, w1, w2):
    i12 = jnp.concatenate([i1, i2], axis=1)
    w12 = jnp.concatenate([w1, w2], axis=1)
    oh = i12[:, :, None] == jnp.arange(E, dtype=jnp.int32)[None, None, :]
    A = oh.sum(1).astype(jnp.int32)
    cum = jnp.cumsum(A, axis=0)
    counts = cum[-1]
    cum_excl = cum - A
    pos = jnp.sum(jnp.where(oh, cum_excl[:, None, :], 0), axis=2)
    tiles = (counts + G - 1) // G
    na = jnp.sum(tiles).astype(jnp.int32)
    tile_start = jnp.concatenate(
        [jnp.zeros((1,), jnp.int32),
         jnp.cumsum(tiles)[:-1].astype(jnp.int32)])
    row = jnp.take(tile_start * G, i12) + pos
    rowf = row.reshape(-1)
    tok = jnp.repeat(jnp.arange(T, dtype=jnp.int32), 2)
    src = (jnp.arange(NTG, dtype=jnp.int32) % T).at[rowf].set(tok)
    wsort = jnp.zeros((NTG,), jnp.float32).at[rowf].set(w12.reshape(-1))
    idx_t = jnp.arange(NT, dtype=jnp.int32)
    te = jnp.searchsorted(tile_start, idx_t, side="right").astype(jnp.int32) - 1
    te = jnp.clip(te, 0, E - 1)
    te = jnp.where(idx_t < na, te, jnp.take(te, na - 1))
    te_ext = jnp.concatenate([te, na[None]])
    return src, wsort.reshape(NTG, 1), te_ext, row[:, 0], row[:, 1]


_GCH = 64
_GNCH = (NTG // NW) // _GCH


def _gather_body(h_hbm, src_hbm, x_hbm, idx0, idx1, buf0, buf1,
                 gs0, gs1, os0, os1):
    wid = lax.axis_index("s") * NC + lax.axis_index("c")
    base = wid * (NTG // NW)
    idx = (idx0, idx1)
    buf = (buf0, buf1)
    gs = (gs0, gs1)
    osem = (os0, os1)

    def issue(c):
        b = base + c * _GCH
        pltpu.sync_copy(src_hbm.at[pl.ds(b, _GCH)], idx[c % 2])
        return [
            pltpu.async_copy(
                h_hbm.at[idx[c % 2].at[pl.ds(q * L, L)]],
                buf[c % 2].at[pl.ds(q * L, L)],
                gs[c % 2])
            for q in range(_GCH // L)
        ]

    gh = {0: issue(0)}
    oh = {}
    for c in range(_GNCH):
        if c + 1 < _GNCH:
            if c - 1 in oh:
                oh[c - 1].wait()
            gh[c + 1] = issue(c + 1)
        for hh in gh[c]:
            hh.wait()
        oh[c] = pltpu.async_copy(
            buf[c % 2], x_hbm.at[pl.ds(base + c * _GCH, _GCH)], osem[c % 2])
    oh[_GNCH - 2].wait()
    oh[_GNCH - 1].wait()


def _gather(h, src):
    mesh = plsc.VectorSubcoreMesh(
        core_axis_name="c", subcore_axis_name="s", num_cores=NC,
        num_subcores=NS)
    f = pl.kernel(
        _gather_body,
        out_type=jax.ShapeDtypeStruct((NTG, D), jnp.float32),
        mesh=mesh,
        scratch_types=[
            pltpu.VMEM((_GCH,), jnp.int32),
            pltpu.VMEM((_GCH,), jnp.int32),
            pltpu.VMEM((_GCH, D), jnp.float32),
            pltpu.VMEM((_GCH, D), jnp.float32),
            pltpu.SemaphoreType.DMA,
            pltpu.SemaphoreType.DMA,
            pltpu.SemaphoreType.DMA,
            pltpu.SemaphoreType.DMA,
        ],
    )
    return f(h, src)


def _gmm_body(te_ref, x_ref, w_ref, wg_ref, wu_ref, wd_ref, y_ref):
    i = pl.program_id(0)

    @pl.when(i < te_ref[NT])
    def _():
        x = x_ref[...]
        g = jnp.dot(x, wg_ref[0], preferred_element_type=jnp.float32)
        u = jnp.dot(x, wu_ref[0], preferred_element_type=jnp.float32)
        a = (g * jax.nn.sigmoid(g)) * u
        y = jnp.dot(a, wd_ref[0], preferred_element_type=jnp.float32)
        y_ref[...] = w_ref[...] * y


def _gmm(te_ext, X, wsort, W_gate, W_up, W_down):
    def _rows(i, te):
        return (jnp.minimum(i, te[NT] - 1), 0)

    def _widx(i, te):
        return (te[jnp.minimum(i, te[NT] - 1)], 0, 0)

    grid_spec = pltpu.PrefetchScalarGridSpec(
        num_scalar_prefetch=1,
        grid=(NT,),
        in_specs=[
            pl.BlockSpec((G, D), _rows),
            pl.BlockSpec((G, 1), _rows),
            pl.BlockSpec((1, D, F), _widx),
            pl.BlockSpec((1, D, F), _widx),
            pl.BlockSpec((1, F, D), _widx),
        ],
        out_specs=pl.BlockSpec((G, D), _rows),
    )
    return pl.pallas_call(
        _gmm_body,
        grid_spec=grid_spec,
        out_shape=jax.ShapeDtypeStruct((NTG, D), jnp.float32),
    )(te_ext, X, wsort, W_gate, W_up, W_down)


_CSUB = (T // NW) // L


def _combine_body(y_hbm, r0_hbm, r1_hbm, res_hbm, out_hbm,
                  idx0a, idx0b, idx1a, idx1b, b0a, b0b, b1a, b1b, bra, brb,
                  sg0a, sg0b, sg1a, sg1b, sra, srb, soa, sob):
    wid = lax.axis_index("s") * NC + lax.axis_index("c")
    per_w = T // NW
    idx0 = (idx0a, idx0b)
    idx1 = (idx1a, idx1b)
    b0 = (b0a, b0b)
    b1 = (b1a, b1b)
    br = (bra, brb)
    sg0 = (sg0a, sg0b)
    sg1 = (sg1a, sg1b)
    sr = (sra, srb)
    so = (soa, sob)

    def issue(s):
        p = s % 2
        t0 = wid * per_w + s * L
        pltpu.sync_copy(r0_hbm.at[pl.ds(t0, L)], idx0[p])
        pltpu.sync_copy(r1_hbm.at[pl.ds(t0, L)], idx1[p])
        return [
            pltpu.async_copy(y_hbm.at[idx0[p]], b0[p], sg0[p]),
            pltpu.async_copy(y_hbm.at[idx1[p]], b1[p], sg1[p]),
            pltpu.async_copy(res_hbm.at[pl.ds(t0, L)], br[p], sr[p]),
        ]

    gh = {0: issue(0)}
    oh = {}
    for s in range(_CSUB):
        p = s % 2
        if s + 1 < _CSUB:
            if s - 1 >= 0:
                oh[s - 1].wait()
            gh[s + 1] = issue(s + 1)
        for hh in gh[s]:
            hh.wait()

        def col(c, r):
            sl = pl.ds(c * L, L)
            br[p][r, sl] = br[p][r, sl] + b0[p][r, sl] + b1[p][r, sl]
            return r

        def rowloop(r, _):
            lax.fori_loop(0, D // L, col, r, unroll=8)
            return 0

        lax.fori_loop(0, L, rowloop, 0)
        oh[s] = pltpu.async_copy(
            br[p], out_hbm.at[pl.ds(wid * per_w + s * L, L)], so[p])
    oh[_CSUB - 2].wait()
    oh[_CSUB - 1].wait()


def _combine(Y, r0, r1, residual):
    mesh = plsc.VectorSubcoreMesh(
        core_axis_name="c", subcore_axis_name="s", num_cores=NC,
        num_subcores=NS)
    f = pl.kernel(
        _combine_body,
        out_type=jax.ShapeDtypeStruct((T, D), jnp.float32),
        mesh=mesh,
        scratch_types=(
            [pltpu.VMEM((L,), jnp.int32)] * 4
            + [pltpu.VMEM((L, D), jnp.float32)] * 6
            + [pltpu.SemaphoreType.DMA] * 8
        ),
    )
    return f(Y, r0, r1, residual)


def kernel(hidden_states, attention_mask, position_ids, ln_weight, router_w,
           W_gate, W_up, W_down):
    x = hidden_states.reshape(T, D)
    h, i1, i2, w1, w2 = _route(x, ln_weight, router_w)
    src, wsort, te_ext, r0, r1 = _metadata(i1, i2, w1, w2)
    X = _gather(h, src)
    Y = _gmm(te_ext, X, wsort, W_gate, W_up, W_down)
    out = _combine(Y, r0, r1, x)
    return out.reshape(B, S, D)


# speedup vs baseline: 3.5928x; 1.5226x over previous
import functools

import jax
import jax.numpy as jnp
from jax import lax
from jax.experimental import pallas as pl
from jax.experimental.pallas import tpu as pltpu
from jax.experimental.pallas import tpu_sc as plsc

B, S, D = 1, 2048, 768
E, K, F = 64, 2, 128
EPS = 1e-6
T = B * S

G = 64
NT = 128
NTG = NT * G
TE_LEN = 136
NC, NS = 2, 16
NW = NC * NS
L = 16


def _plan_body(x_ref, lnw_ref, rw_ref,
               h_ref, r0_ref, r1_ref, w1_ref, w2_ref, te_ref):
    x = x_ref[...]
    v = jnp.mean(x * x, axis=-1, keepdims=True)
    h = x * jax.lax.rsqrt(v + EPS) * lnw_ref[...]
    h_ref[...] = h
    logits = jnp.dot(h, rw_ref[...], preferred_element_type=jnp.float32)
    iota_e = jax.lax.broadcasted_iota(jnp.int32, logits.shape, 1)
    m1 = jnp.max(logits, axis=-1, keepdims=True)
    i1 = jnp.min(jnp.where(logits == m1, iota_e, E), axis=-1, keepdims=True)
    l2 = jnp.where(iota_e == i1, -jnp.inf, logits)
    m2 = jnp.max(l2, axis=-1, keepdims=True)
    i2 = jnp.min(jnp.where(l2 == m2, iota_e, E), axis=-1, keepdims=True)
    e2 = jnp.exp(m2 - m1)
    s = 1.0 + e2
    w1_ref[...] = 1.0 / s
    w2_ref[...] = e2 / s

    oh1 = iota_e == i1
    oh2 = iota_e == i2
    A = oh1.astype(jnp.float32) + oh2.astype(jnp.float32)
    ri = jax.lax.broadcasted_iota(jnp.int32, (T, T), 0)
    ci = jax.lax.broadcasted_iota(jnp.int32, (T, T), 1)
    lower = (ci < ri).astype(jnp.float32)
    cum_excl = jnp.dot(lower, A, preferred_element_type=jnp.float32)
    counts = jnp.sum(A, axis=0, keepdims=True)
    tiles = jnp.floor((counts + (G - 1)) / G)
    ue = jax.lax.broadcasted_iota(jnp.int32, (E, E), 0)
    ve = jax.lax.broadcasted_iota(jnp.int32, (E, E), 1)
    upper = (ue < ve).astype(jnp.float32)
    tstart = jnp.dot(tiles, upper, preferred_element_type=jnp.float32)
    na = jnp.sum(tiles)
    start_rows = tstart * G
    pos1 = jnp.sum(jnp.where(oh1, cum_excl, 0.0), axis=1, keepdims=True)
    base1 = jnp.sum(jnp.where(oh1, start_rows, 0.0), axis=1, keepdims=True)
    r0_ref[...] = (base1 + pos1).astype(jnp.int32)
    pos2 = jnp.sum(jnp.where(oh2, cum_excl, 0.0), axis=1, keepdims=True)
    base2 = jnp.sum(jnp.where(oh2, start_rows, 0.0), axis=1, keepdims=True)
    r1_ref[...] = (base2 + pos2).astype(jnp.int32)

    ti = jax.lax.broadcasted_iota(jnp.int32, (TE_LEN, E), 0).astype(
        jnp.float32)
    te_val = jnp.sum((tstart <= ti).astype(jnp.int32), axis=1,
                     keepdims=True) - 1
    te_val = jnp.clip(te_val, 0, E - 1)
    last_te = jnp.sum((tstart <= (na - 1.0)).astype(jnp.int32)) - 1
    iv = jax.lax.broadcasted_iota(jnp.int32, (TE_LEN, 1), 0)
    te_full = jnp.where(iv.astype(jnp.float32) < na, te_val, last_te)
    te_ref[...] = jnp.where(iv == NT, na.astype(jnp.int32), te_full)


def _plan(x, ln_weight, router_w):
    return pl.pallas_call(
        _plan_body,
        out_shape=(
            jax.ShapeDtypeStruct((T, D), jnp.float32),
            jax.ShapeDtypeStruct((T, 1), jnp.int32),
            jax.ShapeDtypeStruct((T, 1), jnp.int32),
            jax.ShapeDtypeStruct((T, 1), jnp.float32),
            jax.ShapeDtypeStruct((T, 1), jnp.float32),
            jax.ShapeDtypeStruct((TE_LEN, 1), jnp.int32),
        ),
    )(x, ln_weight.reshape(1, D), router_w)


def _scatter_body(h_hbm, r0_hbm, r1_hbm, x_hbm, idx0, idx1, buf, s0, s1, sb):
    wid = lax.axis_index("s") * NC + lax.axis_index("c")
    per_w = T // NW
    tb = wid * per_w
    pltpu.sync_copy(r0_hbm.at[pl.ds(tb, per_w)], idx0)
    pltpu.sync_copy(r1_hbm.at[pl.ds(tb, per_w)], idx1)
    pltpu.async_copy(h_hbm.at[pl.ds(tb, per_w)], buf, sb).wait()
    c0 = pltpu.async_copy(buf, x_hbm.at[idx0], s0)
    c1 = pltpu.async_copy(buf, x_hbm.at[idx1], s1)
    c0.wait()
    c1.wait()


def _scatter(h, r0, r1):
    mesh = plsc.VectorSubcoreMesh(
        core_axis_name="c", subcore_axis_name="s", num_cores=NC,
        num_subcores=NS)
    f = pl.kernel(
        _scatter_body,
        out_type=jax.ShapeDtypeStruct((NTG, D), jnp.float32),
        mesh=mesh,
        scratch_types=[
            pltpu.VMEM((T // NW,), jnp.int32),
            pltpu.VMEM((T // NW,), jnp.int32),
            pltpu.VMEM((T // NW, D), jnp.float32),
            pltpu.SemaphoreType.DMA,
            pltpu.SemaphoreType.DMA,
            pltpu.SemaphoreType.DMA,
        ],
    )
    return f(h, r0, r1)


def _gmm_body(te_ref, x_ref, wg_ref, wu_ref, wd_ref, y_ref):
    i = pl.program_id(0)

    @pl.when(i < te_ref[NT])
    def _():
        x = x_ref[...]
        g = jnp.dot(x, wg_ref[0], preferred_element_type=jnp.float32)
        u = jnp.dot(x, wu_ref[0], preferred_element_type=jnp.float32)
        a = (g * jax.nn.sigmoid(g)) * u
        y_ref[...] = jnp.dot(a, wd_ref[0], preferred_element_type=jnp.float32)


def _gmm(te_ext, X, W_gate, W_up, W_down):
    def _rows(i, te):
        return (jnp.minimum(i, te[NT] - 1), 0)

    def _widx(i, te):
        return (te[jnp.minimum(i, te[NT] - 1)], 0, 0)

    grid_spec = pltpu.PrefetchScalarGridSpec(
        num_scalar_prefetch=1,
        grid=(NT,),
        in_specs=[
            pl.BlockSpec((G, D), _rows),
            pl.BlockSpec((1, D, F), _widx),
            pl.BlockSpec((1, D, F), _widx),
            pl.BlockSpec((1, F, D), _widx),
        ],
        out_specs=pl.BlockSpec((G, D), _rows),
    )
    return pl.pallas_call(
        _gmm_body,
        grid_spec=grid_spec,
        out_shape=jax.ShapeDtypeStruct((NTG, D), jnp.float32),
    )(te_ext, X, W_gate, W_up, W_down)


_CSUB = (T // NW) // L


def _combine_body(y_hbm, r0_hbm, r1_hbm, w1_hbm, w2_hbm, res_hbm, out_hbm,
                  idx0a, idx0b, idx1a, idx1b, w0a, w0b, w1a, w1b,
                  b0a, b0b, b1a, b1b, bra, brb,
                  sg0a, sg0b, sg1a, sg1b, sra, srb, soa, sob):
    wid = lax.axis_index("s") * NC + lax.axis_index("c")
    per_w = T // NW
    idx0 = (idx0a, idx0b)
    idx1 = (idx1a, idx1b)
    wb0 = (w0a, w0b)
    wb1 = (w1a, w1b)
    b0 = (b0a, b0b)
    b1 = (b1a, b1b)
    br = (bra, brb)
    sg0 = (sg0a, sg0b)
    sg1 = (sg1a, sg1b)
    sr = (sra, srb)
    so = (soa, sob)

    def issue(s):
        p = s % 2
        t0 = wid * per_w + s * L
        pltpu.sync_copy(r0_hbm.at[pl.ds(t0, L)], idx0[p])
        pltpu.sync_copy(r1_hbm.at[pl.ds(t0, L)], idx1[p])
        pltpu.sync_copy(w1_hbm.at[pl.ds(t0, L)], wb0[p])
        pltpu.sync_copy(w2_hbm.at[pl.ds(t0, L)], wb1[p])
        return [
            pltpu.async_copy(y_hbm.at[idx0[p]], b0[p], sg0[p]),
            pltpu.async_copy(y_hbm.at[idx1[p]], b1[p], sg1[p]),
            pltpu.async_copy(res_hbm.at[pl.ds(t0, L)], br[p], sr[p]),
        ]

    gh = {0: issue(0)}
    oh = {}
    for s in range(_CSUB):
        p = s % 2
        if s + 1 < _CSUB:
            if s - 1 in oh:
                oh[s - 1].wait()
            gh[s + 1] = issue(s + 1)
        for hh in gh[s]:
            hh.wait()

        wfull0 = wb0[p][...]
        wfull1 = wb1[p][...]
        for r in range(L):
            wv0 = wfull0[r]
            wv1 = wfull1[r]

            def col(c, r_, wv0=wv0, wv1=wv1, r_s=r):
                sl = pl.ds(c * L, L)
                br[p][r_s, sl] = (br[p][r_s, sl] + wv0 * b0[p][r_s, sl]
                                  + wv1 * b1[p][r_s, sl])
                return r_

            lax.fori_loop(0, D // L, col, 0, unroll=8)
        oh[s] = pltpu.async_copy(
            br[p], out_hbm.at[pl.ds(wid * per_w + s * L, L)], so[p])
    oh[_CSUB - 2].wait()
    oh[_CSUB - 1].wait()


def _combine(Y, r0, r1, w1, w2, residual):
    mesh = plsc.VectorSubcoreMesh(
        core_axis_name="c", subcore_axis_name="s", num_cores=NC,
        num_subcores=NS)
    f = pl.kernel(
        _combine_body,
        out_type=jax.ShapeDtypeStruct((T, D), jnp.float32),
        mesh=mesh,
        scratch_types=(
            [pltpu.VMEM((L,), jnp.int32)] * 4
            + [pltpu.VMEM((L,), jnp.float32)] * 4
            + [pltpu.VMEM((L, D), jnp.float32)] * 6
            + [pltpu.SemaphoreType.DMA] * 8
        ),
    )
    return f(Y, r0, r1, w1, w2, residual)


def kernel(hidden_states, attention_mask, position_ids, ln_weight, router_w,
           W_gate, W_up, W_down):
    x = hidden_states.reshape(T, D)
    h, r0, r1, w1, w2, te = _plan(x, ln_weight, router_w)
    r0 = r0.reshape(T)
    r1 = r1.reshape(T)
    te_ext = te.reshape(TE_LEN)
    X = _scatter(h, r0, r1)
    Y = _gmm(te_ext, X, W_gate, W_up, W_down)
    out = _combine(Y, r0, r1, w1.reshape(T), w2.reshape(T), x)
    return out.reshape(B, S, D)


# speedup vs baseline: 3.6556x; 1.0175x over previous
import functools

import jax
import jax.numpy as jnp
from jax import lax
from jax.experimental import pallas as pl
from jax.experimental.pallas import tpu as pltpu
from jax.experimental.pallas import tpu_sc as plsc

B, S, D = 1, 2048, 768
E, K, F = 64, 2, 128
EPS = 1e-6
T = B * S

G = 64
NT = 128
NTG = NT * G
TE_LEN = 136
NC, NS = 2, 16
NW = NC * NS
L = 16


def _plan_body(x_ref, lnw_ref, rw_ref,
               h_ref, r0_ref, r1_ref, w1_ref, w2_ref, te_ref):
    x = x_ref[...]
    v = jnp.mean(x * x, axis=-1, keepdims=True)
    h = x * jax.lax.rsqrt(v + EPS) * lnw_ref[...]
    h_ref[...] = h
    logits = jnp.dot(h, rw_ref[...], preferred_element_type=jnp.float32)
    iota_e = jax.lax.broadcasted_iota(jnp.int32, logits.shape, 1)
    m1 = jnp.max(logits, axis=-1, keepdims=True)
    i1 = jnp.min(jnp.where(logits == m1, iota_e, E), axis=-1, keepdims=True)
    l2 = jnp.where(iota_e == i1, -jnp.inf, logits)
    m2 = jnp.max(l2, axis=-1, keepdims=True)
    i2 = jnp.min(jnp.where(l2 == m2, iota_e, E), axis=-1, keepdims=True)
    e2 = jnp.exp(m2 - m1)
    s = 1.0 + e2
    w1_ref[...] = 1.0 / s
    w2_ref[...] = e2 / s

    oh1 = iota_e == i1
    oh2 = iota_e == i2
    A = oh1.astype(jnp.float32) + oh2.astype(jnp.float32)
    ri = jax.lax.broadcasted_iota(jnp.int32, (T, T), 0)
    ci = jax.lax.broadcasted_iota(jnp.int32, (T, T), 1)
    lower = (ci < ri).astype(jnp.float32)
    cum_excl = jnp.dot(lower, A, preferred_element_type=jnp.float32)
    counts = jnp.sum(A, axis=0, keepdims=True)
    tiles = jnp.floor((counts + (G - 1)) / G)
    ue = jax.lax.broadcasted_iota(jnp.int32, (E, E), 0)
    ve = jax.lax.broadcasted_iota(jnp.int32, (E, E), 1)
    upper = (ue < ve).astype(jnp.float32)
    tstart = jnp.dot(tiles, upper, preferred_element_type=jnp.float32)
    na = jnp.sum(tiles)
    start_rows = tstart * G
    pos1 = jnp.sum(jnp.where(oh1, cum_excl, 0.0), axis=1, keepdims=True)
    base1 = jnp.sum(jnp.where(oh1, start_rows, 0.0), axis=1, keepdims=True)
    r0_ref[...] = (base1 + pos1).astype(jnp.int32)
    pos2 = jnp.sum(jnp.where(oh2, cum_excl, 0.0), axis=1, keepdims=True)
    base2 = jnp.sum(jnp.where(oh2, start_rows, 0.0), axis=1, keepdims=True)
    r1_ref[...] = (base2 + pos2).astype(jnp.int32)

    ti = jax.lax.broadcasted_iota(jnp.int32, (TE_LEN, E), 0).astype(
        jnp.float32)
    te_val = jnp.sum((tstart <= ti).astype(jnp.int32), axis=1,
                     keepdims=True) - 1
    te_val = jnp.clip(te_val, 0, E - 1)
    last_te = jnp.sum((tstart <= (na - 1.0)).astype(jnp.int32)) - 1
    iv = jax.lax.broadcasted_iota(jnp.int32, (TE_LEN, 1), 0)
    te_full = jnp.where(iv.astype(jnp.float32) < na, te_val, last_te)
    te_ref[...] = jnp.where(iv == NT, na.astype(jnp.int32), te_full)


def _plan(x, ln_weight, router_w):
    return pl.pallas_call(
        _plan_body,
        out_shape=(
            jax.ShapeDtypeStruct((T, D), jnp.float32),
            jax.ShapeDtypeStruct((T, 1), jnp.int32),
            jax.ShapeDtypeStruct((T, 1), jnp.int32),
            jax.ShapeDtypeStruct((T, 1), jnp.float32),
            jax.ShapeDtypeStruct((T, 1), jnp.float32),
            jax.ShapeDtypeStruct((TE_LEN, 1), jnp.int32),
        ),
    )(x, ln_weight.reshape(1, D), router_w)


def _scatter_body(h_hbm, r0_hbm, r1_hbm, x_hbm, idx0, idx1, buf, s0, s1, sb):
    wid = lax.axis_index("s") * NC + lax.axis_index("c")
    per_w = T // NW
    tb = wid * per_w
    pltpu.sync_copy(r0_hbm.at[pl.ds(tb, per_w)], idx0)
    pltpu.sync_copy(r1_hbm.at[pl.ds(tb, per_w)], idx1)
    pltpu.async_copy(h_hbm.at[pl.ds(tb, per_w)], buf, sb).wait()
    c0 = pltpu.async_copy(buf, x_hbm.at[idx0], s0)
    c1 = pltpu.async_copy(buf, x_hbm.at[idx1], s1)
    c0.wait()
    c1.wait()


def _scatter(h, r0, r1):
    mesh = plsc.VectorSubcoreMesh(
        core_axis_name="c", subcore_axis_name="s", num_cores=NC,
        num_subcores=NS)
    f = pl.kernel(
        _scatter_body,
        out_type=jax.ShapeDtypeStruct((NTG, D), jnp.float32),
        mesh=mesh,
        scratch_types=[
            pltpu.VMEM((T // NW,), jnp.int32),
            pltpu.VMEM((T // NW,), jnp.int32),
            pltpu.VMEM((T // NW, D), jnp.float32),
            pltpu.SemaphoreType.DMA,
            pltpu.SemaphoreType.DMA,
            pltpu.SemaphoreType.DMA,
        ],
    )
    return f(h, r0, r1)


def _gmm_body(te_ref, x_ref, wg_ref, wu_ref, wd_ref, y_ref):
    i = pl.program_id(0)

    @pl.when(i < te_ref[NT])
    def _():
        x = x_ref[...]
        g = jnp.dot(x, wg_ref[0], preferred_element_type=jnp.float32)
        u = jnp.dot(x, wu_ref[0], preferred_element_type=jnp.float32)
        a = (g * jax.nn.sigmoid(g)) * u
        y_ref[...] = jnp.dot(a, wd_ref[0], preferred_element_type=jnp.float32)


def _gmm(te_ext, X, W_gate, W_up, W_down):
    def _rows(i, te):
        return (jnp.minimum(i, te[NT] - 1), 0)

    def _widx(i, te):
        return (te[jnp.minimum(i, te[NT] - 1)], 0, 0)

    grid_spec = pltpu.PrefetchScalarGridSpec(
        num_scalar_prefetch=1,
        grid=(NT,),
        in_specs=[
            pl.BlockSpec((G, D), _rows),
            pl.BlockSpec((1, D, F), _widx),
            pl.BlockSpec((1, D, F), _widx),
            pl.BlockSpec((1, F, D), _widx),
        ],
        out_specs=pl.BlockSpec((G, D), _rows),
    )
    return pl.pallas_call(
        _gmm_body,
        grid_spec=grid_spec,
        out_shape=jax.ShapeDtypeStruct((NTG, D), jnp.float32),
    )(te_ext, X, W_gate, W_up, W_down)


_CSUB = (T // NW) // L


def _combine_body(y_hbm, r0_hbm, r1_hbm, w1_hbm, w2_hbm, res_hbm, out_hbm,
                  idx0, idx1, wb0, wb1,
                  b0a, b0b, b1a, b1b, bra, brb,
                  sg0a, sg0b, sg1a, sg1b, sra, srb, soa, sob):
    wid = lax.axis_index("s") * NC + lax.axis_index("c")
    per_w = T // NW
    b0 = (b0a, b0b)
    b1 = (b1a, b1b)
    br = (bra, brb)
    sg0 = (sg0a, sg0b)
    sg1 = (sg1a, sg1b)
    sr = (sra, srb)
    so = (soa, sob)

    tb = wid * per_w
    pltpu.sync_copy(r0_hbm.at[pl.ds(tb, per_w)], idx0)
    pltpu.sync_copy(r1_hbm.at[pl.ds(tb, per_w)], idx1)
    pltpu.sync_copy(w1_hbm.at[pl.ds(tb, per_w)], wb0)
    pltpu.sync_copy(w2_hbm.at[pl.ds(tb, per_w)], wb1)

    def issue(s):
        p = s % 2
        t0 = tb + s * L
        return [
            pltpu.async_copy(y_hbm.at[idx0.at[pl.ds(s * L, L)]], b0[p],
                             sg0[p]),
            pltpu.async_copy(y_hbm.at[idx1.at[pl.ds(s * L, L)]], b1[p],
                             sg1[p]),
            pltpu.async_copy(res_hbm.at[pl.ds(t0, L)], br[p], sr[p]),
        ]

    gh = {0: issue(0)}
    oh = {}
    for s in range(_CSUB):
        p = s % 2
        if s + 1 < _CSUB:
            if s - 1 in oh:
                oh[s - 1].wait()
            gh[s + 1] = issue(s + 1)
        for hh in gh[s]:
            hh.wait()

        wfull0 = wb0[pl.ds(s * L, L)]
        wfull1 = wb1[pl.ds(s * L, L)]
        for r in range(L):
            wv0 = wfull0[r]
            wv1 = wfull1[r]

            def col(c, r_, wv0=wv0, wv1=wv1, r_s=r):
                sl = pl.ds(c * L, L)
                br[p][r_s, sl] = (br[p][r_s, sl] + wv0 * b0[p][r_s, sl]
                                  + wv1 * b1[p][r_s, sl])
                return r_

            lax.fori_loop(0, D // L, col, 0, unroll=8)
        oh[s] = pltpu.async_copy(
            br[p], out_hbm.at[pl.ds(tb + s * L, L)], so[p])
    oh[_CSUB - 2].wait()
    oh[_CSUB - 1].wait()


def _combine(Y, r0, r1, w1, w2, residual):
    mesh = plsc.VectorSubcoreMesh(
        core_axis_name="c", subcore_axis_name="s", num_cores=NC,
        num_subcores=NS)
    f = pl.kernel(
        _combine_body,
        out_type=jax.ShapeDtypeStruct((T, D), jnp.float32),
        mesh=mesh,
        scratch_types=(
            [pltpu.VMEM((T // NW,), jnp.int32)] * 2
            + [pltpu.VMEM((T // NW,), jnp.float32)] * 2
            + [pltpu.VMEM((L, D), jnp.float32)] * 6
            + [pltpu.SemaphoreType.DMA] * 8
        ),
    )
    return f(Y, r0, r1, w1, w2, residual)


def kernel(hidden_states, attention_mask, position_ids, ln_weight, router_w,
           W_gate, W_up, W_down):
    x = hidden_states.reshape(T, D)
    h, r0, r1, w1, w2, te = _plan(x, ln_weight, router_w)
    r0 = r0.reshape(T)
    r1 = r1.reshape(T)
    te_ext = te.reshape(TE_LEN)
    X = _scatter(h, r0, r1)
    Y = _gmm(te_ext, X, W_gate, W_up, W_down)
    out = _combine(Y, r0, r1, w1.reshape(T), w2.reshape(T), x)
    return out.reshape(B, S, D)
